# baseline (speedup 1.0000x reference)
import functools

import jax
import jax.numpy as jnp
from jax import lax
from jax.experimental import pallas as pl
from jax.experimental.pallas import tpu as pltpu
from jax.experimental.pallas import tpu_sc as plsc

BIGRAM_VOCAB = 100000
HID = 128
MODEL_DIM = 1024
BATCH = 4
SEQLEN = 4096
TOK = BATCH * SEQLEN

NC, NS = 2, 16
NW = NC * NS
GSTREAM = 128


def _make_sc_gather(ctok):
    chunk = ctok // NW
    ng = -(-chunk // GSTREAM)
    gs = chunk // ng
    vecs = chunk // 16

    @functools.partial(
        pl.kernel,
        mesh=plsc.VectorSubcoreMesh(core_axis_name="c", subcore_axis_name="s"),
        out_type=jax.ShapeDtypeStruct((ctok, HID), jnp.float32),
        scratch_types=[
            pltpu.VMEM((chunk,), jnp.int32),
            pltpu.VMEM((chunk,), jnp.int32),
            pltpu.VMEM((ng, gs), jnp.int32),
            pltpu.VMEM((chunk, HID), jnp.float32),
            pltpu.SemaphoreType.DMA,
            pltpu.SemaphoreType.DMA,
            pltpu.SemaphoreType.DMA,
        ],
    )
    def sc_gather(cur_hbm, prev_hbm, table_hbm, h_hbm, cur_v, prev_v, idx_v,
                  rows_v, sem_in, sem_g, sem_w):
        wid = lax.axis_index("s") * NC + lax.axis_index("c")
        base = wid * chunk
        ld_cur = pltpu.async_copy(cur_hbm.at[pl.ds(base, chunk)], cur_v,
                                  sem_in)
        ld_prev = pltpu.async_copy(prev_hbm.at[pl.ds(base, chunk)], prev_v,
                                   sem_in)
        ld_cur.wait()
        ld_prev.wait()
        vecs_per_g = gs // 16
        gathers = []
        for j in range(ng):
            for v in range(vecs_per_g):
                i = j * vecs_per_g + v
                cur = cur_v[pl.ds(i * 16, 16)]
                prev = prev_v[pl.ds(i * 16, 16)]
                h = lax.rem(prev * 3 + cur, jnp.int32(100000))
                idx_v[j, pl.ds(v * 16, 16)] = h
            gathers.append(
                pltpu.async_copy(table_hbm.at[idx_v.at[jnp.int32(j)]],
                                 rows_v.at[pl.ds(j * gs, gs)], sem_g))
        writes = []
        for j in range(ng):
            gathers[j].wait()
            writes.append(
                pltpu.async_copy(rows_v.at[pl.ds(j * gs, gs)],
                                 h_hbm.at[pl.ds(base + j * gs, gs)], sem_w))
        for cp in writes:
            cp.wait()

    return sc_gather


_sc_gather_chunk = _make_sc_gather(TOK)


def _proj_body(h_ref, w_ref, o_ref):
    o_ref[...] = lax.dot_general(
        h_ref[...].astype(jnp.bfloat16), w_ref[...].astype(jnp.bfloat16),
        (((1,), (1,)), ((), ())),
        preferred_element_type=jnp.float32)


_ROWS_BLK = 2048


def _tc_project(h, Wproj):
    rows = h.shape[0]
    return pl.pallas_call(
        _proj_body,
        grid=(rows // _ROWS_BLK,),
        in_specs=[
            pl.BlockSpec((_ROWS_BLK, HID), lambda i: (i, jnp.int32(0))),
            pl.BlockSpec((MODEL_DIM, HID),
                         lambda i: (jnp.int32(0), jnp.int32(0))),
        ],
        out_specs=pl.BlockSpec((_ROWS_BLK, MODEL_DIM),
                               lambda i: (i, jnp.int32(0))),
        out_shape=jax.ShapeDtypeStruct((rows, MODEL_DIM), jnp.float32),
    )(h, Wproj)


def kernel(input_ids, table, Wproj):
    ids32 = input_ids.astype(jnp.int32)
    prev32 = jnp.concatenate(
        [jnp.zeros((BATCH, 1), jnp.int32), ids32[:, :-1]], axis=1)
    h = _sc_gather_chunk(ids32.reshape(TOK), prev32.reshape(TOK), table)
    out = _tc_project(h, Wproj)
    return out.reshape(BATCH, SEQLEN, MODEL_DIM)


# speedup vs baseline: 1.0195x; 1.0195x over previous
import functools

import jax
import jax.numpy as jnp
from jax import lax
from jax.experimental import pallas as pl
from jax.experimental.pallas import tpu as pltpu
from jax.experimental.pallas import tpu_sc as plsc

BIGRAM_VOCAB = 100000
HID = 128
MODEL_DIM = 1024
BATCH = 4
SEQLEN = 4096
TOK = BATCH * SEQLEN

NC, NS = 2, 16
NW = NC * NS
GSTREAM = 64


def _make_sc_gather(ctok):
    chunk = ctok // NW
    ng = -(-chunk // GSTREAM)
    gs = chunk // ng
    vecs = chunk // 16

    @functools.partial(
        pl.kernel,
        mesh=plsc.VectorSubcoreMesh(core_axis_name="c", subcore_axis_name="s"),
        out_type=jax.ShapeDtypeStruct((ctok, HID), jnp.float32),
        scratch_types=[
            pltpu.VMEM((chunk,), jnp.int32),
            pltpu.VMEM((chunk,), jnp.int32),
            pltpu.VMEM((ng, gs), jnp.int32),
            pltpu.VMEM((chunk, HID), jnp.float32),
            pltpu.SemaphoreType.DMA,
            pltpu.SemaphoreType.DMA,
            pltpu.SemaphoreType.DMA,
        ],
    )
    def sc_gather(cur_hbm, prev_hbm, table_hbm, h_hbm, cur_v, prev_v, idx_v,
                  rows_v, sem_in, sem_g, sem_w):
        wid = lax.axis_index("s") * NC + lax.axis_index("c")
        base = wid * chunk
        ld_cur = pltpu.async_copy(cur_hbm.at[pl.ds(base, chunk)], cur_v,
                                  sem_in)
        ld_prev = pltpu.async_copy(prev_hbm.at[pl.ds(base, chunk)], prev_v,
                                   sem_in)
        ld_cur.wait()
        ld_prev.wait()
        vecs_per_g = gs // 16
        gathers = []
        for j in range(ng):
            for v in range(vecs_per_g):
                i = j * vecs_per_g + v
                cur = cur_v[pl.ds(i * 16, 16)]
                prev = prev_v[pl.ds(i * 16, 16)]
                h = lax.rem(prev * 3 + cur, jnp.int32(100000))
                idx_v[j, pl.ds(v * 16, 16)] = h
            gathers.append(
                pltpu.async_copy(table_hbm.at[idx_v.at[jnp.int32(j)]],
                                 rows_v.at[pl.ds(j * gs, gs)], sem_g))
        writes = []
        for j in range(ng):
            gathers[j].wait()
            writes.append(
                pltpu.async_copy(rows_v.at[pl.ds(j * gs, gs)],
                                 h_hbm.at[pl.ds(base + j * gs, gs)], sem_w))
        for cp in writes:
            cp.wait()

    return sc_gather


_sc_gather_chunk = _make_sc_gather(TOK)


def _proj_body(h_ref, w_ref, o_ref):
    o_ref[...] = lax.dot_general(
        h_ref[...], w_ref[...], (((1,), (1,)), ((), ())),
        preferred_element_type=jnp.float32)


_ROWS_BLK = 2048


def _tc_project(h, Wproj):
    rows = h.shape[0]
    return pl.pallas_call(
        _proj_body,
        grid=(rows // _ROWS_BLK,),
        in_specs=[
            pl.BlockSpec((_ROWS_BLK, HID), lambda i: (i, jnp.int32(0))),
            pl.BlockSpec((MODEL_DIM, HID),
                         lambda i: (jnp.int32(0), jnp.int32(0))),
        ],
        out_specs=pl.BlockSpec((_ROWS_BLK, MODEL_DIM),
                               lambda i: (i, jnp.int32(0))),
        out_shape=jax.ShapeDtypeStruct((rows, MODEL_DIM), jnp.float32),
    )(h, Wproj)


def kernel(input_ids, table, Wproj):
    ids32 = input_ids.astype(jnp.int32)
    prev32 = jnp.concatenate(
        [jnp.zeros((BATCH, 1), jnp.int32), ids32[:, :-1]], axis=1)
    h = _sc_gather_chunk(ids32.reshape(TOK), prev32.reshape(TOK), table)
    out = _tc_project(h, Wproj)
    return out.reshape(BATCH, SEQLEN, MODEL_DIM)


# speedup vs baseline: 1.0434x; 1.0234x over previous
import functools

import jax
import jax.numpy as jnp
from jax import lax
from jax.experimental import pallas as pl
from jax.experimental.pallas import tpu as pltpu
from jax.experimental.pallas import tpu_sc as plsc

BIGRAM_VOCAB = 100000
HID = 128
MODEL_DIM = 1024
BATCH = 4
SEQLEN = 4096
TOK = BATCH * SEQLEN

NC, NS = 2, 16
NW = NC * NS
GSTREAM = 32


def _make_sc_gather(ctok):
    chunk = ctok // NW
    ng = -(-chunk // GSTREAM)
    gs = chunk // ng
    vecs = chunk // 16

    @functools.partial(
        pl.kernel,
        mesh=plsc.VectorSubcoreMesh(core_axis_name="c", subcore_axis_name="s"),
        out_type=jax.ShapeDtypeStruct((ctok, HID), jnp.float32),
        scratch_types=[
            pltpu.VMEM((chunk,), jnp.int32),
            pltpu.VMEM((chunk,), jnp.int32),
            pltpu.VMEM((ng, gs), jnp.int32),
            pltpu.VMEM((chunk, HID), jnp.float32),
            pltpu.SemaphoreType.DMA,
            pltpu.SemaphoreType.DMA,
            pltpu.SemaphoreType.DMA,
        ],
    )
    def sc_gather(cur_hbm, prev_hbm, table_hbm, h_hbm, cur_v, prev_v, idx_v,
                  rows_v, sem_in, sem_g, sem_w):
        wid = lax.axis_index("s") * NC + lax.axis_index("c")
        base = wid * chunk
        ld_cur = pltpu.async_copy(cur_hbm.at[pl.ds(base, chunk)], cur_v,
                                  sem_in)
        ld_prev = pltpu.async_copy(prev_hbm.at[pl.ds(base, chunk)], prev_v,
                                   sem_in)
        ld_cur.wait()
        ld_prev.wait()
        vecs_per_g = gs // 16
        gathers = []
        for j in range(ng):
            for v in range(vecs_per_g):
                i = j * vecs_per_g + v
                cur = cur_v[pl.ds(i * 16, 16)]
                prev = prev_v[pl.ds(i * 16, 16)]
                h = lax.rem(prev * 3 + cur, jnp.int32(100000))
                idx_v[j, pl.ds(v * 16, 16)] = h
            gathers.append(
                pltpu.async_copy(table_hbm.at[idx_v.at[jnp.int32(j)]],
                                 rows_v.at[pl.ds(j * gs, gs)], sem_g))
        writes = []
        for j in range(ng):
            gathers[j].wait()
            writes.append(
                pltpu.async_copy(rows_v.at[pl.ds(j * gs, gs)],
                                 h_hbm.at[pl.ds(base + j * gs, gs)], sem_w))
        for cp in writes:
            cp.wait()

    return sc_gather


_sc_gather_chunk = _make_sc_gather(TOK)


def _proj_body(h_ref, w_ref, o_ref):
    o_ref[...] = lax.dot_general(
        h_ref[...], w_ref[...], (((1,), (1,)), ((), ())),
        preferred_element_type=jnp.float32)


_ROWS_BLK = 2048


def _tc_project(h, Wproj):
    rows = h.shape[0]
    return pl.pallas_call(
        _proj_body,
        grid=(rows // _ROWS_BLK,),
        in_specs=[
            pl.BlockSpec((_ROWS_BLK, HID), lambda i: (i, jnp.int32(0))),
            pl.BlockSpec((MODEL_DIM, HID),
                         lambda i: (jnp.int32(0), jnp.int32(0))),
        ],
        out_specs=pl.BlockSpec((_ROWS_BLK, MODEL_DIM),
                               lambda i: (i, jnp.int32(0))),
        out_shape=jax.ShapeDtypeStruct((rows, MODEL_DIM), jnp.float32),
    )(h, Wproj)


def kernel(input_ids, table, Wproj):
    ids32 = input_ids.astype(jnp.int32)
    prev32 = jnp.concatenate(
        [jnp.zeros((BATCH, 1), jnp.int32), ids32[:, :-1]], axis=1)
    h = _sc_gather_chunk(ids32.reshape(TOK), prev32.reshape(TOK), table)
    out = _tc_project(h, Wproj)
    return out.reshape(BATCH, SEQLEN, MODEL_DIM)
